# Initial kernel scaffold; baseline (speedup 1.0000x reference)
import jax
import jax.numpy as jnp
from jax.experimental import pallas as pl


def kernel(outputs, default_bboxes, gt_boxes, gt_classes):
    raise NotImplementedError("write your pallas kernel here")


# speedup vs baseline: 57.4463x; 57.4463x over previous
import functools

import jax
import jax.numpy as jnp
from jax.experimental import pallas as pl
from jax.experimental.pallas import tpu as pltpu

_N = 8732
_G = 16
_C = 21
_B = 8
_P1 = 72
_P2 = 128
_NPAD = _P1 * _P2
_MAXF_BITS = 0x7F7FFFFF


def _huber(x):
    ax = jnp.abs(x)
    return jnp.where(ax < 1.0, 0.5 * x * x, ax - 0.5)


def _topk_sum(vi, k, iters=31):
    lo = jnp.zeros((_B, 1, 1), jnp.int32)
    hi = jnp.full((_B, 1, 1), _MAXF_BITS, jnp.int32)

    def body(_, lohi):
        lo, hi = lohi
        mid = lo + ((hi - lo + 1) >> 1)
        cnt = jnp.sum(jnp.where(vi >= mid, 1.0, 0.0), axis=(1, 2),
                      keepdims=True)
        ge = cnt >= k
        return jnp.where(ge, mid, lo), jnp.where(ge, hi, mid - 1)

    lo, hi = jax.lax.fori_loop(0, iters, body, (lo, hi))
    t = lo
    tv = jax.lax.bitcast_convert_type(t, jnp.float32)
    vf = jax.lax.bitcast_convert_type(vi, jnp.float32)
    sel_gt = vi > t
    sum_gt = jnp.sum(jnp.where(sel_gt, vf, 0.0), axis=(1, 2), keepdims=True)
    cnt_gt = jnp.sum(jnp.where(sel_gt, 1.0, 0.0), axis=(1, 2), keepdims=True)
    return sum_gt + (k - cnt_gt) * tv


def _ssd_body(gtb_ref, gtc_ref, outs_ref, db_ref, out_ref,
              delta_s, pos_s, lse_s, vn_s, vp_s):
    f32 = jnp.float32
    cx = db_ref[0]
    cy = db_ref[1]
    w = db_ref[2]
    h = db_ref[3]
    x1 = cx - w * 0.5
    x2 = cx + w * 0.5
    y1 = cy - h * 0.5
    y2 = cy + h * 0.5
    area_d = w * h

    nidx = (jax.lax.broadcasted_iota(jnp.int32, (_P1, _P2), 0) * _P2
            + jax.lax.broadcasted_iota(jnp.int32, (_P1, _P2), 1))
    valid = nidx < _N

    pos_any = jnp.zeros((_P1, _P2), f32)
    np_cnt = jnp.float32(0.0)
    for g in range(_G):
        gcx = gtb_ref[0, g]
        gcy = gtb_ref[1, g]
        gw = gtb_ref[2, g]
        gh = gtb_ref[3, g]
        gx1 = gcx - gw * 0.5
        gx2 = gcx + gw * 0.5
        gy1 = gcy - gh * 0.5
        gy2 = gcy + gh * 0.5
        iw = jnp.minimum(x2, gx2) - jnp.maximum(x1, gx1)
        ih = jnp.minimum(y2, gy2) - jnp.maximum(y1, gy1)
        inter = jnp.where((iw > 0.0) & (ih > 0.0), iw * ih, 0.0)
        iou = inter / (area_d + gw * gh - inter)
        posg = jnp.where((iou > 0.5) & valid, 1.0, 0.0)
        pos_s[g] = posg
        pos_any = jnp.maximum(pos_any, posg)
        np_cnt = np_cnt + jnp.sum(posg)
        delta_s[0, g] = (gcx - cx) / w
        delta_s[1, g] = (gcy - cy) / h
        delta_s[2, g] = jnp.log(gw / w)
        delta_s[3, g] = jnp.log(gh / h)

    negf = jnp.where((pos_any == 0.0) & valid, 1.0, 0.0)
    nneg = jnp.sum(negf)
    cond = np_cnt * 3.0 > nneg
    nn = jnp.where(cond, nneg, np_cnt * 3.0)

    conf_pos_total = jnp.float32(0.0)
    loc_total = jnp.float32(0.0)
    for b in range(_B):
        rows = [outs_ref[b, 4 + c] for c in range(_C)]
        m = rows[0]
        for c in range(1, _C):
            m = jnp.maximum(m, rows[c])
        s = jnp.exp(rows[0] - m)
        for c in range(1, _C):
            s = s + jnp.exp(rows[c] - m)
        lse = m + jnp.log(s)
        lse_s[b] = lse
        l0 = outs_ref[b, 0]
        l1 = outs_ref[b, 1]
        l2 = outs_ref[b, 2]
        l3 = outs_ref[b, 3]
        for g in range(_G):
            cls = gtc_ref[g]
            logit_g = outs_ref[b, 4 + cls]
            posg = pos_s[g]
            conf = lse - logit_g
            conf_pos_total = conf_pos_total + jnp.sum(
                jnp.where(posg > 0.0, conf, 0.0))
            sl1 = (_huber(l0 - delta_s[0, g]) + _huber(l1 - delta_s[1, g])
                   + _huber(l2 - delta_s[2, g]) + _huber(l3 - delta_s[3, g]))
            loc_total = loc_total + jnp.sum(
                jnp.where(posg > 0.0, sl1, 0.0))
        vneg = lse - outs_ref[b, 8]
        vn_s[b] = jnp.where(
            negf > 0.0, jax.lax.bitcast_convert_type(vneg, jnp.int32), -1)

    neg_sums = _topk_sum(vn_s[...], nn)
    neg_total = jnp.sum(neg_sums)

    inv = 1.0 / (np_cnt * _B)
    out_ref[0, 0] = (loc_total + conf_pos_total + neg_total) * inv

    @pl.when(cond)
    def _rare():
        pn = jnp.floor(nneg / 3.0)
        for g in range(_G):
            cls = gtc_ref[g]
            conf = lse_s[...] - outs_ref[:, 4 + cls]
            vp_s[g] = jnp.where(
                pos_s[g][None] > 0.0,
                jax.lax.bitcast_convert_type(conf, jnp.int32), -1)

        lo = jnp.zeros((_B, 1, 1), jnp.int32)
        hi = jnp.full((_B, 1, 1), _MAXF_BITS, jnp.int32)

        def body(_, lohi):
            lo, hi = lohi
            mid = lo + ((hi - lo + 1) >> 1)
            cnt = jnp.zeros((_B, 1, 1), f32)
            for g in range(_G):
                cnt = cnt + jnp.sum(
                    jnp.where(vp_s[g] >= mid, 1.0, 0.0), axis=(1, 2),
                    keepdims=True)
            ge = cnt >= pn
            return jnp.where(ge, mid, lo), jnp.where(ge, hi, mid - 1)

        lo, hi = jax.lax.fori_loop(0, 31, body, (lo, hi))
        t = lo
        tv = jax.lax.bitcast_convert_type(t, jnp.float32)
        conf_gt = jnp.zeros((_B, 1, 1), f32)
        cnt_gt = jnp.zeros((_B, 1, 1), f32)
        loc_gt = jnp.zeros((_B, 1, 1), f32)
        cnt_eq = jnp.zeros((_B, 1, 1), f32)
        loc_eq = jnp.zeros((_B, 1, 1), f32)
        for g in range(_G):
            vp = vp_s[g]
            vf = jax.lax.bitcast_convert_type(vp, jnp.float32)
            sl1 = (_huber(outs_ref[:, 0] - delta_s[0, g][None])
                   + _huber(outs_ref[:, 1] - delta_s[1, g][None])
                   + _huber(outs_ref[:, 2] - delta_s[2, g][None])
                   + _huber(outs_ref[:, 3] - delta_s[3, g][None]))
            sgt = vp > t
            seq = vp == t
            conf_gt = conf_gt + jnp.sum(jnp.where(sgt, vf, 0.0), axis=(1, 2),
                                        keepdims=True)
            cnt_gt = cnt_gt + jnp.sum(jnp.where(sgt, 1.0, 0.0), axis=(1, 2),
                                      keepdims=True)
            loc_gt = loc_gt + jnp.sum(jnp.where(sgt, sl1, 0.0), axis=(1, 2),
                                      keepdims=True)
            cnt_eq = cnt_eq + jnp.sum(jnp.where(seq, 1.0, 0.0), axis=(1, 2),
                                      keepdims=True)
            loc_eq = loc_eq + jnp.sum(jnp.where(seq, sl1, 0.0), axis=(1, 2),
                                      keepdims=True)
        rem = pn - cnt_gt
        conf_rare = jnp.sum(conf_gt + rem * tv)
        loc_rare = jnp.sum(
            loc_gt + rem * loc_eq / jnp.maximum(cnt_eq, 1.0))
        out_ref[0, 0] = (loc_rare + conf_rare + neg_total) * inv


def kernel(outputs, default_bboxes, gt_boxes, gt_classes):
    f32 = jnp.float32
    outs = jnp.transpose(outputs.astype(f32), (0, 2, 1))
    outs = jnp.pad(outs, ((0, 0), (0, 0), (0, _NPAD - _N)))
    outs = outs.reshape(_B, _C + 4, _P1, _P2)
    pad = jnp.broadcast_to(jnp.array([10.0, 10.0, 1.0, 1.0], f32),
                           (_NPAD - _N, 4))
    db = jnp.concatenate([default_bboxes.astype(f32), pad], axis=0)
    db = db.T.reshape(4, _P1, _P2)
    gtb = gt_boxes.astype(f32).T
    gtc = gt_classes.astype(jnp.int32)

    out = pl.pallas_call(
        _ssd_body,
        out_shape=jax.ShapeDtypeStruct((1, 1), f32),
        in_specs=[
            pl.BlockSpec(memory_space=pltpu.SMEM),
            pl.BlockSpec(memory_space=pltpu.SMEM),
            pl.BlockSpec(memory_space=pltpu.VMEM),
            pl.BlockSpec(memory_space=pltpu.VMEM),
        ],
        out_specs=pl.BlockSpec(memory_space=pltpu.SMEM),
        scratch_shapes=[
            pltpu.VMEM((4, _G, _P1, _P2), f32),
            pltpu.VMEM((_G, _P1, _P2), f32),
            pltpu.VMEM((_B, _P1, _P2), f32),
            pltpu.VMEM((_B, _P1, _P2), jnp.int32),
            pltpu.VMEM((_G, _B, _P1, _P2), jnp.int32),
        ],
    )(gtb, gtc, outs, db)
    return out[0, 0]
